# baseline (speedup 1.0000x reference)
import functools

import jax
import jax.numpy as jnp
from jax import lax
from jax.experimental import pallas as pl
from jax.experimental.pallas import tpu as pltpu
from jax.experimental.pallas import tpu_sc as plsc

NC, NS = 2, 16
NW = NC * NS
LANES = 16
ROWS_PER_CHUNK = 4


FOLD_CHUNK = 160


def kernel(input, weight):
    B0, B1 = input.shape
    V, D = weight.shape
    H = V // 2
    D2 = 2 * D
    rpw = B0 // NW
    W = ROWS_PER_CHUNK * B1
    cpw = rpw // ROWS_PER_CHUNK
    n_chunks = B0 // ROWS_PER_CHUNK

    idx = input.reshape(n_chunks, W).astype(jnp.int32)

    mesh = plsc.VectorSubcoreMesh(core_axis_name="c", subcore_axis_name="s")

    fc = FOLD_CHUNK
    nfc = H // fc
    iters = -(-nfc // NW)
    iters += iters % 2

    @functools.partial(
        pl.kernel,
        mesh=mesh,
        out_type=jax.ShapeDtypeStruct((H, D2), jnp.float32),
        scratch_types=[
            pltpu.VMEM((2, fc, D), jnp.float32),
            pltpu.VMEM((2, fc, D), jnp.float32),
            pltpu.VMEM((2, fc, D2), jnp.float32),
            pltpu.SemaphoreType.DMA,
            pltpu.SemaphoreType.DMA,
            pltpu.SemaphoreType.DMA,
            pltpu.SemaphoreType.DMA,
        ],
    )
    def sc_fold(w_hbm, slab_hbm, buf_t, buf_b, buf, i0, i1, o0, o1):
        wid = lax.axis_index("s") * NC + lax.axis_index("c")
        isems = (i0, i1)
        osems = (o0, o1)

        def start_in(t, p):
            c = wid + NW * t

            @pl.when(c < nfc)
            def _():
                k0 = c * fc
                pltpu.async_copy(w_hbm.at[pl.ds(k0, fc)],
                                 buf_t.at[p], isems[p])
                pltpu.async_copy(w_hbm.at[pl.ds(H + k0, fc)],
                                 buf_b.at[p], isems[p])

        def drain(t, p):
            c = wid + NW * t

            @pl.when(c < nfc)
            def _():
                pltpu.make_async_copy(
                    w_hbm.at[pl.ds(0, fc)], buf_t.at[p], isems[p]).wait()
                pltpu.make_async_copy(
                    w_hbm.at[pl.ds(0, fc)], buf_b.at[p], isems[p]).wait()

                @pl.loop(0, fc)
                def _(r):
                    for k in range(D // LANES):
                        sl = pl.ds(k * LANES, LANES)
                        buf[p, r, pl.ds(k * LANES, LANES)] = buf_t[p, r, sl]
                        buf[p, r, pl.ds(D + k * LANES, LANES)] = (
                            buf_b[p, r, sl])

                pltpu.sync_copy(buf.at[p], slab_hbm.at[pl.ds(c * fc, fc)])

        start_in(0, 0)
        start_in(1, 1)

        @pl.loop(0, iters // 2)
        def _(h):
            for p in range(2):
                t = 2 * h + p
                drain(t, p)
                start_in(t + 2, p)

    slabs = sc_fold(weight)

    starts = list(range(0, W - LANES + 1, LANES))
    if starts[-1] != W - LANES:
        starts.append(W - LANES)
    prev_end = 0
    groups = []
    for s in starts:
        groups.append((s, prev_end - s))
        prev_end = s + LANES

    @functools.partial(
        pl.kernel,
        mesh=mesh,
        out_type=jax.ShapeDtypeStruct((B0, B1, D), jnp.float32),
        scratch_types=[
            pltpu.VMEM((cpw, W), jnp.int32),
            pltpu.VMEM((2, W), jnp.int32),
            pltpu.VMEM((2, W, D2), jnp.float32),
            pltpu.VMEM((2, ROWS_PER_CHUNK, B1, D), jnp.float32),
            pltpu.SemaphoreType.DMA,
            pltpu.SemaphoreType.DMA,
            pltpu.SemaphoreType.DMA,
            pltpu.SemaphoreType.DMA,
        ],
    )
    def sc_gather(table_hbm, idx_hbm, out_hbm, idx_v, slab_v, rows_v, out_v,
                  g0, g1, w0, w1):
        wid = lax.axis_index("s") * NC + lax.axis_index("c")
        cbase = wid * cpw
        rbase = wid * rpw
        pltpu.sync_copy(idx_hbm.at[pl.ds(cbase, cpw)], idx_v)

        gsems = (g0, g1)
        wsems = (w0, w1)

        def compute_slabs(t, p):
            for gstart, _ in groups:
                sl = pl.ds(gstart, LANES)
                iv = idx_v[t, sl]
                slab_v[p, sl] = jnp.where(iv >= H, iv - H, iv)

        def start_gather(t, p):
            pltpu.async_copy(table_hbm.at[slab_v.at[p]], rows_v.at[p],
                             gsems[p])

        def wait_gather(p):
            pltpu.make_async_copy(table_hbm.at[slab_v.at[p]], rows_v.at[p],
                                  gsems[p]).wait()

        def start_write(t, p):
            pltpu.async_copy(
                out_v.at[p],
                out_hbm.at[pl.ds(rbase + t * ROWS_PER_CHUNK, ROWS_PER_CHUNK)],
                wsems[p])

        def wait_write(p):
            pltpu.make_async_copy(
                out_v.at[p], out_hbm.at[pl.ds(0, ROWS_PER_CHUNK)],
                wsems[p]).wait()

        def select(t, p):
            for gstart, lane0 in groups:
                offv = jnp.where(idx_v[t, pl.ds(gstart, LANES)] >= H, D, 0)
                for lane in range(lane0, LANES):
                    r = gstart + lane
                    q, rr = divmod(r, B1)
                    off = offv[lane]
                    for k in range(D // LANES):
                        out_v[p, q, rr, pl.ds(k * LANES, LANES)] = (
                            rows_v[p, r, pl.ds(off + k * LANES, LANES)])

        compute_slabs(0, 0)
        start_gather(0, 0)
        compute_slabs(1, 1)
        start_gather(1, 1)

        @pl.loop(0, cpw // 2)
        def _(h):
            for p in range(2):
                t = 2 * h + p
                wait_gather(p)

                @pl.when(h > 0)
                def _():
                    wait_write(p)

                select(t, p)
                start_write(t, p)

                @pl.when(h < cpw // 2 - 1)
                def _():
                    compute_slabs(t + 2, p)
                    start_gather(t + 2, p)

        wait_write(0)
        wait_write(1)

    return sc_gather(slabs, idx)


# speedup vs baseline: 1.1356x; 1.1356x over previous
import functools

import jax
import jax.numpy as jnp
from jax import lax
from jax.experimental import pallas as pl
from jax.experimental.pallas import tpu as pltpu
from jax.experimental.pallas import tpu_sc as plsc

NC, NS = 2, 16
NW = NC * NS
LANES = 16
FOLD_BLOCK = 4000
ROWS_PER_CHUNK = 4


def _fold_body(x_ref, o_ref):
    d = x_ref.shape[2]
    o_ref[:, :d] = x_ref[0]
    o_ref[:, d:] = x_ref[1]


def kernel(input, weight):
    B0, B1 = input.shape
    V, D = weight.shape
    H = V // 2
    D2 = 2 * D
    rpw = B0 // NW
    W = ROWS_PER_CHUNK * B1
    cpw = rpw // ROWS_PER_CHUNK
    n_chunks = B0 // ROWS_PER_CHUNK

    idx = input.reshape(n_chunks, W).astype(jnp.int32)

    slabs = pl.pallas_call(
        _fold_body,
        grid=(H // FOLD_BLOCK,),
        in_specs=[
            pl.BlockSpec((2, FOLD_BLOCK, D), lambda i: (0, i, 0)),
        ],
        out_specs=pl.BlockSpec((FOLD_BLOCK, D2), lambda i: (i, 0)),
        out_shape=jax.ShapeDtypeStruct((H, D2), jnp.float32),
    )(weight.reshape(2, H, D))

    mesh = plsc.VectorSubcoreMesh(core_axis_name="c", subcore_axis_name="s")

    starts = list(range(0, W - LANES + 1, LANES))
    if starts[-1] != W - LANES:
        starts.append(W - LANES)
    prev_end = 0
    groups = []
    for s in starts:
        groups.append((s, prev_end - s))
        prev_end = s + LANES

    @functools.partial(
        pl.kernel,
        mesh=mesh,
        out_type=jax.ShapeDtypeStruct((B0, B1, D), jnp.float32),
        scratch_types=[
            pltpu.VMEM((cpw, W), jnp.int32),
            pltpu.VMEM((4, W), jnp.int32),
            pltpu.VMEM((4, W, D2), jnp.float32),
            pltpu.VMEM((2, ROWS_PER_CHUNK, B1, D), jnp.float32),
            pltpu.SemaphoreType.DMA,
            pltpu.SemaphoreType.DMA,
            pltpu.SemaphoreType.DMA,
            pltpu.SemaphoreType.DMA,
            pltpu.SemaphoreType.DMA,
            pltpu.SemaphoreType.DMA,
        ],
    )
    def sc_gather(table_hbm, idx_hbm, out_hbm, idx_v, slab_v, rows_v, out_v,
                  g0, g1, g2, g3, w0, w1):
        wid = lax.axis_index("s") * NC + lax.axis_index("c")
        cbase = wid * cpw
        rbase = wid * rpw
        pltpu.sync_copy(idx_hbm.at[pl.ds(cbase, cpw)], idx_v)

        gsems = (g0, g1, g2, g3)
        wsems = (w0, w1)

        def compute_slabs(t, p):
            for gstart, _ in groups:
                sl = pl.ds(gstart, LANES)
                iv = idx_v[t, sl]
                slab_v[p, sl] = jnp.where(iv >= H, iv - H, iv)

        def start_gather(t, p):
            pltpu.async_copy(table_hbm.at[slab_v.at[p]], rows_v.at[p],
                             gsems[p])

        def wait_gather(p):
            pltpu.make_async_copy(table_hbm.at[slab_v.at[p]], rows_v.at[p],
                                  gsems[p]).wait()

        def start_write(t, q):
            pltpu.async_copy(
                out_v.at[q],
                out_hbm.at[pl.ds(rbase + t * ROWS_PER_CHUNK, ROWS_PER_CHUNK)],
                wsems[q])

        def wait_write(q):
            pltpu.make_async_copy(
                out_v.at[q], out_hbm.at[pl.ds(0, ROWS_PER_CHUNK)],
                wsems[q]).wait()

        def select(t, p, q):
            for gstart, lane0 in groups:
                offv = jnp.where(idx_v[t, pl.ds(gstart, LANES)] >= H, D, 0)
                for lane in range(lane0, LANES):
                    r = gstart + lane
                    qq, rr = divmod(r, B1)
                    off = offv[lane]
                    for k in range(D // LANES):
                        out_v[q, qq, rr, pl.ds(k * LANES, LANES)] = (
                            rows_v[p, r, pl.ds(off + k * LANES, LANES)])

        for p in range(4):
            compute_slabs(p, p)
            start_gather(p, p)

        @pl.loop(0, cpw // 4)
        def _(h):
            for p in range(4):
                t = 4 * h + p
                q = p % 2
                wait_gather(p)

                @pl.when((h > 0) | (p >= 2))
                def _():
                    wait_write(q)

                select(t, p, q)
                start_write(t, q)

                @pl.when(h < cpw // 4 - 1)
                def _():
                    compute_slabs(t + 4, p)
                    start_gather(t + 4, p)

        for q in range(2):
            wait_write(q)

    return sc_gather(slabs, idx)


# speedup vs baseline: 1.1582x; 1.0198x over previous
import functools

import jax
import jax.numpy as jnp
from jax import lax
from jax.experimental import pallas as pl
from jax.experimental.pallas import tpu as pltpu
from jax.experimental.pallas import tpu_sc as plsc

NC, NS = 2, 16
NW = NC * NS
LANES = 16
FOLD_BLOCK = 4000
ROWS_PER_CHUNK = 4


def _fold_body(x_ref, o_ref):
    d = x_ref.shape[2]
    o_ref[:, :d] = x_ref[0]
    o_ref[:, d:] = x_ref[1]


def kernel(input, weight):
    B0, B1 = input.shape
    V, D = weight.shape
    H = V // 2
    D2 = 2 * D
    rpw = B0 // NW
    W = ROWS_PER_CHUNK * B1
    cpw = rpw // ROWS_PER_CHUNK
    n_chunks = B0 // ROWS_PER_CHUNK

    idx = input.reshape(n_chunks, W).astype(jnp.int32)

    slabs = pl.pallas_call(
        _fold_body,
        grid=(H // FOLD_BLOCK,),
        in_specs=[
            pl.BlockSpec((2, FOLD_BLOCK, D), lambda i: (0, i, 0)),
        ],
        out_specs=pl.BlockSpec((FOLD_BLOCK, D2), lambda i: (i, 0)),
        out_shape=jax.ShapeDtypeStruct((H, D2), jnp.float32),
    )(weight.reshape(2, H, D))

    mesh = plsc.VectorSubcoreMesh(core_axis_name="c", subcore_axis_name="s")

    starts = list(range(0, W - LANES + 1, LANES))
    if starts[-1] != W - LANES:
        starts.append(W - LANES)
    prev_end = 0
    groups = []
    for s in starts:
        groups.append((s, prev_end - s))
        prev_end = s + LANES

    @functools.partial(
        pl.kernel,
        mesh=mesh,
        out_type=jax.ShapeDtypeStruct((B0, B1, D), jnp.float32),
        scratch_types=[
            pltpu.VMEM((cpw, W), jnp.int32),
            pltpu.VMEM((2, W), jnp.int32),
            pltpu.VMEM((2, W, D2), jnp.float32),
            pltpu.VMEM((2, ROWS_PER_CHUNK, B1, D), jnp.float32),
            pltpu.SemaphoreType.DMA,
            pltpu.SemaphoreType.DMA,
            pltpu.SemaphoreType.DMA,
            pltpu.SemaphoreType.DMA,
        ],
    )
    def sc_gather(table_hbm, idx_hbm, out_hbm, idx_v, slab_v, rows_v, out_v,
                  g0, g1, w0, w1):
        wid = lax.axis_index("s") * NC + lax.axis_index("c")
        cbase = wid * cpw
        rbase = wid * rpw
        pltpu.sync_copy(idx_hbm.at[pl.ds(cbase, cpw)], idx_v)

        gsems = (g0, g1)
        wsems = (w0, w1)

        def compute_slabs(t, p):
            for gstart, _ in groups:
                sl = pl.ds(gstart, LANES)
                iv = idx_v[t, sl]
                slab_v[p, sl] = jnp.where(iv >= H, iv - H, iv)

        def start_gather(t, p):
            pltpu.async_copy(table_hbm.at[slab_v.at[p]], rows_v.at[p],
                             gsems[p])

        def wait_gather(p):
            pltpu.make_async_copy(table_hbm.at[slab_v.at[p]], rows_v.at[p],
                                  gsems[p]).wait()

        def start_write(t, p):
            pltpu.async_copy(
                out_v.at[p],
                out_hbm.at[pl.ds(rbase + t * ROWS_PER_CHUNK, ROWS_PER_CHUNK)],
                wsems[p])

        def wait_write(p):
            pltpu.make_async_copy(
                out_v.at[p], out_hbm.at[pl.ds(0, ROWS_PER_CHUNK)],
                wsems[p]).wait()

        def select(t, p):
            for gstart, lane0 in groups:
                offv = jnp.where(idx_v[t, pl.ds(gstart, LANES)] >= H, D, 0)
                for lane in range(lane0, LANES):
                    r = gstart + lane
                    q, rr = divmod(r, B1)
                    off = offv[lane]
                    for k in range(D // LANES):
                        out_v[p, q, rr, pl.ds(k * LANES, LANES)] = (
                            rows_v[p, r, pl.ds(off + k * LANES, LANES)])

        compute_slabs(0, 0)
        start_gather(0, 0)
        compute_slabs(1, 1)
        start_gather(1, 1)

        @pl.loop(0, cpw // 2)
        def _(h):
            for p in range(2):
                t = 2 * h + p
                wait_gather(p)

                @pl.when(h > 0)
                def _():
                    wait_write(p)

                select(t, p)
                start_write(t, p)

                @pl.when(h < cpw // 2 - 1)
                def _():
                    compute_slabs(t + 2, p)
                    start_gather(t + 2, p)

        wait_write(0)
        wait_write(1)

    return sc_gather(slabs, idx)
